# Initial kernel scaffold; baseline (speedup 1.0000x reference)
import jax
import jax.numpy as jnp
from jax.experimental import pallas as pl


def kernel(input_seq_batch, seq_lengths, table):
    raise NotImplementedError("write your pallas kernel here")


# speedup vs baseline: 2.5844x; 2.5844x over previous
import functools

import jax
import jax.numpy as jnp
from jax import lax
from jax.experimental import pallas as pl
from jax.experimental.pallas import tpu as pltpu
from jax.experimental.pallas import tpu_sc as plsc

_B = 16384
_L = 50
_D = 32
_NC = 2
_NS = 16
_NW = _NC * _NS
_BPW = _B // _NW
_CB = 64
_NCH = _BPW // _CB
_RPD = 100
_DPC = _CB * _L // _RPD
_ROWS = _CB * _L


def _body(idx_hbm, len_hbm, table_hbm, out_hbm,
          idx_v, rows_v, len_v, inv_v, out_v, sem):
    c = lax.axis_index("c")
    s = lax.axis_index("s")
    wid = s * _NC + c
    base_b = wid * _BPW

    pltpu.sync_copy(len_hbm.at[pl.ds(base_b, _BPW)], len_v)

    def inv_body(g, carry):
        lv = len_v[pl.ds(g * 16, 16)]
        inv_v[pl.ds(g * 16, 16)] = 1.0 / lv.astype(jnp.float32)
        return carry

    lax.fori_loop(0, _BPW // 16, inv_body, 0)

    def chunk_body(ch, carry):
        idx_row0 = wid * (_BPW * _L // _RPD) + ch * _DPC
        pltpu.sync_copy(idx_hbm.at[pl.ds(idx_row0, _DPC)], idx_v)
        copies = [
            pltpu.async_copy(table_hbm.at[idx_v.at[j]],
                             rows_v.at[pl.ds(j * _RPD, _RPD)], sem)
            for j in range(_DPC)
        ]
        for cp in copies:
            cp.wait()

        def group_body(g, gcarry):
            lv = inv_v[pl.ds((ch * (_CB // 16) + g) * 16, 16)]
            for t in range(16):
                b = g * 16 + t
                r0 = b * _L
                a0 = rows_v[r0, pl.ds(0, 16)]
                a1 = rows_v[r0, pl.ds(16, 16)]
                for j in range(1, _L):
                    a0 = a0 + rows_v[r0 + j, pl.ds(0, 16)]
                    a1 = a1 + rows_v[r0 + j, pl.ds(16, 16)]
                inv_s = lv[t]
                out_v[b, pl.ds(0, 16)] = a0 * inv_s
                out_v[b, pl.ds(16, 16)] = a1 * inv_s
            return gcarry

        lax.fori_loop(0, _CB // 16, group_body, 0)
        pltpu.sync_copy(out_v, out_hbm.at[pl.ds(base_b + ch * _CB, _CB)])
        return carry

    lax.fori_loop(0, _NCH, chunk_body, 0)


@jax.jit
def kernel(input_seq_batch, seq_lengths, table):
    idx2d = input_seq_batch.reshape(_B * _L // _RPD, _RPD)
    mesh = plsc.VectorSubcoreMesh(core_axis_name="c", subcore_axis_name="s")
    f = pl.kernel(
        _body,
        out_type=jax.ShapeDtypeStruct((_B, _D), jnp.float32),
        mesh=mesh,
        compiler_params=pltpu.CompilerParams(use_tc_tiling_on_sc=False),
        scratch_types=[
            pltpu.VMEM((_DPC, _RPD), jnp.int32),
            pltpu.VMEM((_ROWS, _D), jnp.float32),
            pltpu.VMEM((_BPW,), jnp.int32),
            pltpu.VMEM((_BPW,), jnp.float32),
            pltpu.VMEM((_CB, _D), jnp.float32),
            pltpu.SemaphoreType.DMA,
        ],
    )
    return f(idx2d, seq_lengths, table)
